# baseline (speedup 1.0000x reference)
import jax
import jax.numpy as jnp
from jax.experimental import pallas as pl
from jax.experimental.pallas import tpu as pltpu

_P = 0.3
_XMIN = 1.0728769e-07
_ALPHA = 1.0868737


def _mask_params(B, H, W):
    key = jax.random.key(42)
    k1, k2, k3, k4 = jax.random.split(key, 4)
    sampled = jax.random.bernoulli(k1, _P, (B,))
    rand_row = jax.random.randint(k2, (), 0, H)
    coin = jax.random.bernoulli(k3, 0.5)
    r = jax.random.uniform(k4, (), dtype=jnp.float32)
    rel = jnp.float32(_XMIN) * (1.0 - r) ** (-1.0 / (jnp.float32(_ALPHA) - 1.0))
    return sampled, rand_row, coin, rel


def _body(x_ref, rf_ref, cf_ref, o_ref):
    o_ref[...] = x_ref[...] * rf_ref[...] * cf_ref[...]


def kernel(forward_input):
    B, C, H, W = forward_input.shape
    sampled, rand_row, coin, rel = _mask_params(B, H, W)
    CH = C * H
    one = jnp.float32(1.0)
    hidx = jnp.arange(CH, dtype=jnp.int32) % H
    rowf = jnp.where(
        (~coin) & sampled[:, None] & (hidx[None, :] == rand_row), rel, one
    )
    colf = jnp.where(
        coin
        & sampled[:, None]
        & (jnp.arange(W, dtype=jnp.int32)[None, :] == rand_row),
        rel,
        one,
    )
    x3 = forward_input.reshape(B, CH, W)
    rowf = rowf.reshape(B, CH, 1)
    colf = colf.reshape(B, 1, W)
    BR = 512
    out = pl.pallas_call(
        _body,
        grid=(B, CH // BR),
        in_specs=[
            pl.BlockSpec((1, BR, W), lambda b, j: (b, j, 0)),
            pl.BlockSpec((1, BR, 1), lambda b, j: (b, j, 0)),
            pl.BlockSpec((1, 1, W), lambda b, j: (b, 0, 0)),
        ],
        out_specs=pl.BlockSpec((1, BR, W), lambda b, j: (b, j, 0)),
        out_shape=jax.ShapeDtypeStruct((B, CH, W), jnp.float32),
        compiler_params=pltpu.CompilerParams(
            dimension_semantics=("parallel", "arbitrary")
        ),
    )(x3, rowf, colf)
    return out.reshape(B, C, H, W)


# speedup vs baseline: 1.4753x; 1.4753x over previous
import jax
import jax.numpy as jnp
from jax.experimental import pallas as pl
from jax.experimental.pallas import tpu as pltpu

_P = 0.3
_XMIN = 1.0728769e-07
_ALPHA = 1.0868737

_BR = 1024


def _mask_params(B, H, W):
    key = jax.random.key(42)
    k1, k2, k3, k4 = jax.random.split(key, 4)
    sampled = jax.random.bernoulli(k1, _P, (B,))
    rand_row = jax.random.randint(k2, (), 0, H)
    coin = jax.random.bernoulli(k3, 0.5)
    r = jax.random.uniform(k4, (), dtype=jnp.float32)
    rel = jnp.float32(_XMIN) * (1.0 - r) ** (-1.0 / (jnp.float32(_ALPHA) - 1.0))
    return sampled, rand_row, coin, rel


def _make_body(H, W):
    def _body(ints_ref, relb_ref, x_ref, o_ref):
        b = pl.program_id(0)
        j = pl.program_id(1)
        r = ints_ref[0]
        coin = ints_ref[1]
        relb = relb_ref[b]
        rows = jax.lax.broadcasted_iota(jnp.int32, (1, _BR, W), 1) + j * _BR
        h = jax.lax.rem(rows, H)
        lanes = jax.lax.broadcasted_iota(jnp.int32, (1, _BR, W), 2)
        idx = jnp.where(coin == 1, lanes, h)
        f = jnp.where(idx == r, relb, jnp.float32(1.0))
        o_ref[...] = x_ref[...] * f

    return _body


def kernel(forward_input):
    B, C, H, W = forward_input.shape
    sampled, rand_row, coin, rel = _mask_params(B, H, W)
    CH = C * H
    ints = jnp.stack([rand_row, coin.astype(jnp.int32)]).astype(jnp.int32)
    relb = jnp.where(sampled, rel, jnp.float32(1.0))
    x3 = forward_input.reshape(B, CH, W)
    out = pl.pallas_call(
        _make_body(H, W),
        grid_spec=pltpu.PrefetchScalarGridSpec(
            num_scalar_prefetch=2,
            grid=(B, CH // _BR),
            in_specs=[
                pl.BlockSpec((1, _BR, W), lambda b, j, *_: (b, j, 0)),
            ],
            out_specs=pl.BlockSpec((1, _BR, W), lambda b, j, *_: (b, j, 0)),
        ),
        out_shape=jax.ShapeDtypeStruct((B, CH, W), jnp.float32),
        compiler_params=pltpu.CompilerParams(
            dimension_semantics=("parallel", "arbitrary")
        ),
    )(ints, relb, x3)
    return out.reshape(B, C, H, W)


# speedup vs baseline: 1.6559x; 1.1224x over previous
import jax
import jax.numpy as jnp
from jax.experimental import pallas as pl
from jax.experimental.pallas import tpu as pltpu

_P = 0.3
_XMIN = 1.0728769e-07
_ALPHA = 1.0868737

_BR = 1152


def _mask_params(B, H, W):
    key = jax.random.key(42)
    k1, k2, k3, k4 = jax.random.split(key, 4)
    sampled = jax.random.bernoulli(k1, _P, (B,))
    rand_row = jax.random.randint(k2, (), 0, H)
    coin = jax.random.bernoulli(k3, 0.5)
    r = jax.random.uniform(k4, (), dtype=jnp.float32)
    rel = jnp.float32(_XMIN) * (1.0 - r) ** (-1.0 / (jnp.float32(_ALPHA) - 1.0))
    return sampled, rand_row, coin, rel


def _make_body(H, W):
    k = _BR // H

    def _body(ints_ref, relb_ref, cf_ref, x_ref, o_ref):
        b = pl.program_id(0)
        r = ints_ref[0]
        coin = ints_ref[1]
        relb = relb_ref[b]

        @pl.when(coin == 0)
        def _row_case():
            o_ref[...] = x_ref[...]
            for m in range(k):
                sl = pl.ds(r + m * H, 1)
                o_ref[0, sl, :] = x_ref[0, sl, :] * relb

        @pl.when(coin == 1)
        def _col_case():
            o_ref[...] = x_ref[...] * cf_ref[...]

    return _body


def kernel(forward_input):
    B, C, H, W = forward_input.shape
    sampled, rand_row, coin, rel = _mask_params(B, H, W)
    CH = C * H
    one = jnp.float32(1.0)
    ints = jnp.stack([rand_row, coin.astype(jnp.int32)]).astype(jnp.int32)
    relb = jnp.where(sampled, rel, one)
    cfb = jnp.where(
        sampled[:, None, None]
        & (jnp.arange(W, dtype=jnp.int32)[None, None, :] == rand_row),
        rel,
        one,
    )
    x3 = forward_input.reshape(B, CH, W)
    out = pl.pallas_call(
        _make_body(H, W),
        grid_spec=pltpu.PrefetchScalarGridSpec(
            num_scalar_prefetch=2,
            grid=(B, CH // _BR),
            in_specs=[
                pl.BlockSpec((1, 1, W), lambda b, j, *_: (b, 0, 0)),
                pl.BlockSpec((1, _BR, W), lambda b, j, *_: (b, j, 0)),
            ],
            out_specs=pl.BlockSpec((1, _BR, W), lambda b, j, *_: (b, j, 0)),
        ),
        out_shape=jax.ShapeDtypeStruct((B, CH, W), jnp.float32),
        compiler_params=pltpu.CompilerParams(
            dimension_semantics=("parallel", "arbitrary")
        ),
    )(ints, relb, cfb, x3)
    return out.reshape(B, C, H, W)


# speedup vs baseline: 1.8390x; 1.1106x over previous
import jax
import jax.numpy as jnp
from jax.experimental import pallas as pl
from jax.experimental.pallas import tpu as pltpu

_P = 0.3
_XMIN = 1.0728769e-07
_ALPHA = 1.0868737

_BR = 2304


def _mask_params(B, H, W):
    key = jax.random.key(42)
    k1, k2, k3, k4 = jax.random.split(key, 4)
    sampled = jax.random.bernoulli(k1, _P, (B,))
    rand_row = jax.random.randint(k2, (), 0, H)
    coin = jax.random.bernoulli(k3, 0.5)
    r = jax.random.uniform(k4, (), dtype=jnp.float32)
    rel = jnp.float32(_XMIN) * (1.0 - r) ** (-1.0 / (jnp.float32(_ALPHA) - 1.0))
    return sampled, rand_row, coin, rel


def _make_body(H, W):
    k = _BR // H

    def _body(ints_ref, relb_ref, cf_ref, x_ref, o_ref):
        b = pl.program_id(0)
        r = ints_ref[0]
        coin = ints_ref[1]
        relb = relb_ref[b]

        @pl.when(coin == 0)
        def _row_case():
            o_ref[...] = x_ref[...]
            for m in range(k):
                sl = pl.ds(r + m * H, 1)
                o_ref[0, sl, :] = x_ref[0, sl, :] * relb

        @pl.when(coin == 1)
        def _col_case():
            o_ref[...] = x_ref[...] * cf_ref[...]

    return _body


def kernel(forward_input):
    B, C, H, W = forward_input.shape
    sampled, rand_row, coin, rel = _mask_params(B, H, W)
    CH = C * H
    one = jnp.float32(1.0)
    ints = jnp.stack([rand_row, coin.astype(jnp.int32)]).astype(jnp.int32)
    relb = jnp.where(sampled, rel, one)
    cfb = jnp.where(
        sampled[:, None, None]
        & (jnp.arange(W, dtype=jnp.int32)[None, None, :] == rand_row),
        rel,
        one,
    )
    x3 = forward_input.reshape(B, CH, W)
    out = pl.pallas_call(
        _make_body(H, W),
        grid_spec=pltpu.PrefetchScalarGridSpec(
            num_scalar_prefetch=2,
            grid=(B, CH // _BR),
            in_specs=[
                pl.BlockSpec((1, 1, W), lambda b, j, *_: (b, 0, 0)),
                pl.BlockSpec((1, _BR, W), lambda b, j, *_: (b, j, 0)),
            ],
            out_specs=pl.BlockSpec((1, _BR, W), lambda b, j, *_: (b, j, 0)),
        ),
        out_shape=jax.ShapeDtypeStruct((B, CH, W), jnp.float32),
        compiler_params=pltpu.CompilerParams(
            dimension_semantics=("parallel", "arbitrary")
        ),
    )(ints, relb, cfb, x3)
    return out.reshape(B, C, H, W)


# speedup vs baseline: 1.8644x; 1.0138x over previous
import jax
import jax.numpy as jnp
from jax.experimental import pallas as pl
from jax.experimental.pallas import tpu as pltpu

_P = 0.3
_XMIN = 1.0728769e-07
_ALPHA = 1.0868737

_BR = 4608


def _mask_params(B, H, W):
    key = jax.random.key(42)
    k1, k2, k3, k4 = jax.random.split(key, 4)
    sampled = jax.random.bernoulli(k1, _P, (B,))
    rand_row = jax.random.randint(k2, (), 0, H)
    coin = jax.random.bernoulli(k3, 0.5)
    r = jax.random.uniform(k4, (), dtype=jnp.float32)
    rel = jnp.float32(_XMIN) * (1.0 - r) ** (-1.0 / (jnp.float32(_ALPHA) - 1.0))
    return sampled, rand_row, coin, rel


def _make_body(H, W):
    k = _BR // H

    def _body(ints_ref, relb_ref, cf_ref, x_ref, o_ref):
        b = pl.program_id(0)
        r = ints_ref[0]
        coin = ints_ref[1]
        relb = relb_ref[b]

        @pl.when(coin == 0)
        def _row_case():
            o_ref[...] = x_ref[...]
            for m in range(k):
                sl = pl.ds(r + m * H, 1)
                o_ref[0, sl, :] = x_ref[0, sl, :] * relb

        @pl.when(coin == 1)
        def _col_case():
            o_ref[...] = x_ref[...] * cf_ref[...]

    return _body


def kernel(forward_input):
    B, C, H, W = forward_input.shape
    sampled, rand_row, coin, rel = _mask_params(B, H, W)
    CH = C * H
    one = jnp.float32(1.0)
    ints = jnp.stack([rand_row, coin.astype(jnp.int32)]).astype(jnp.int32)
    relb = jnp.where(sampled, rel, one)
    cfb = jnp.where(
        sampled[:, None, None]
        & (jnp.arange(W, dtype=jnp.int32)[None, None, :] == rand_row),
        rel,
        one,
    )
    x3 = forward_input.reshape(B, CH, W)
    out = pl.pallas_call(
        _make_body(H, W),
        grid_spec=pltpu.PrefetchScalarGridSpec(
            num_scalar_prefetch=2,
            grid=(B, CH // _BR),
            in_specs=[
                pl.BlockSpec((1, 1, W), lambda b, j, *_: (b, 0, 0)),
                pl.BlockSpec((1, _BR, W), lambda b, j, *_: (b, j, 0)),
            ],
            out_specs=pl.BlockSpec((1, _BR, W), lambda b, j, *_: (b, j, 0)),
        ),
        out_shape=jax.ShapeDtypeStruct((B, CH, W), jnp.float32),
        compiler_params=pltpu.CompilerParams(
            dimension_semantics=("parallel", "arbitrary")
        ),
    )(ints, relb, cfb, x3)
    return out.reshape(B, C, H, W)


# speedup vs baseline: 1.8750x; 1.0057x over previous
import jax
import jax.numpy as jnp
from jax.experimental import pallas as pl
from jax.experimental.pallas import tpu as pltpu

_P = 0.3
_XMIN = 1.0728769e-07
_ALPHA = 1.0868737

_BR = 9216


def _mask_params(B, H, W):
    key = jax.random.key(42)
    k1, k2, k3, k4 = jax.random.split(key, 4)
    sampled = jax.random.bernoulli(k1, _P, (B,))
    rand_row = jax.random.randint(k2, (), 0, H)
    coin = jax.random.bernoulli(k3, 0.5)
    r = jax.random.uniform(k4, (), dtype=jnp.float32)
    rel = jnp.float32(_XMIN) * (1.0 - r) ** (-1.0 / (jnp.float32(_ALPHA) - 1.0))
    return sampled, rand_row, coin, rel


def _make_body(H, W):
    k = _BR // H

    def _body(ints_ref, relb_ref, cf_ref, x_ref, o_ref):
        b = pl.program_id(0)
        r = ints_ref[0]
        coin = ints_ref[1]
        relb = relb_ref[b]

        @pl.when(coin == 0)
        def _row_case():
            o_ref[...] = x_ref[...]
            for m in range(k):
                sl = pl.ds(r + m * H, 1)
                o_ref[0, sl, :] = x_ref[0, sl, :] * relb

        @pl.when(coin == 1)
        def _col_case():
            o_ref[...] = x_ref[...] * cf_ref[...]

    return _body


def kernel(forward_input):
    B, C, H, W = forward_input.shape
    sampled, rand_row, coin, rel = _mask_params(B, H, W)
    CH = C * H
    one = jnp.float32(1.0)
    ints = jnp.stack([rand_row, coin.astype(jnp.int32)]).astype(jnp.int32)
    relb = jnp.where(sampled, rel, one)
    cfb = jnp.where(
        sampled[:, None, None]
        & (jnp.arange(W, dtype=jnp.int32)[None, None, :] == rand_row),
        rel,
        one,
    )
    x3 = forward_input.reshape(B, CH, W)
    out = pl.pallas_call(
        _make_body(H, W),
        grid_spec=pltpu.PrefetchScalarGridSpec(
            num_scalar_prefetch=2,
            grid=(B, CH // _BR),
            in_specs=[
                pl.BlockSpec((1, 1, W), lambda b, j, *_: (b, 0, 0)),
                pl.BlockSpec((1, _BR, W), lambda b, j, *_: (b, j, 0)),
            ],
            out_specs=pl.BlockSpec((1, _BR, W), lambda b, j, *_: (b, j, 0)),
        ),
        out_shape=jax.ShapeDtypeStruct((B, CH, W), jnp.float32),
        compiler_params=pltpu.CompilerParams(
            dimension_semantics=("parallel", "arbitrary")
        ),
    )(ints, relb, cfb, x3)
    return out.reshape(B, C, H, W)
